# baseline (speedup 1.0000x reference)
import functools

import jax
import jax.numpy as jnp
from jax import lax
from jax.experimental import pallas as pl
from jax.experimental.pallas import tpu as pltpu
from jax.experimental.pallas import tpu_sc as plsc

B = 16384
L = 50
LPAD = 56
D = 64
NCLS = 100


def _sc_pool(x_pad, table):
  info = plsc.get_sparse_core_info()
  nc, ns = info.num_cores, info.num_subcores
  nw = nc * ns
  spw = B // nw
  cpw = spw // 2
  nbuf = 4
  span = LPAD + L
  x_flat = x_pad.reshape(nw, spw * LPAD)
  mesh = plsc.VectorSubcoreMesh(core_axis_name="c", subcore_axis_name="s")

  @functools.partial(
      pl.kernel,
      out_type=jax.ShapeDtypeStruct((B, D), jnp.float32),
      mesh=mesh,
      scratch_types=[
          pltpu.VMEM((spw * LPAD,), jnp.int32),
          pltpu.VMEM((nbuf, span, D), jnp.float32),
          pltpu.VMEM((spw, D), jnp.float32),
          pltpu.SemaphoreType.DMA((nbuf,)),
      ],
      compiler_params=pltpu.CompilerParams(use_tc_tiling_on_sc=False),
  )
  def k(x_hbm, table_hbm, out_hbm, idx_v, rows_v, pooled_v, sems):
    wid = lax.axis_index("s") * nc + lax.axis_index("c")
    base = wid * spw
    pltpu.sync_copy(x_hbm.at[wid], idx_v)

    def start(c, b):
      off = pl.multiple_of(c * (2 * LPAD), 8)
      pltpu.async_copy(
          table_hbm.at[idx_v.at[pl.ds(off, span)]], rows_v.at[b], sems.at[b])

    def wait(b):
      pltpu.make_async_copy(
          table_hbm.at[pl.ds(0, span)], rows_v.at[b], sems.at[b]).wait()

    def accum(c, b):
      for half, row0 in ((0, 0), (1, LPAD)):
        accs = [rows_v[b, row0, pl.ds(16 * q, 16)] for q in range(D // 16)]
        for j in range(1, L):
          for q in range(D // 16):
            accs[q] = accs[q] + rows_v[b, row0 + j, pl.ds(16 * q, 16)]
        s = 2 * c + half
        for q in range(D // 16):
          pooled_v[s, pl.ds(16 * q, 16)] = accs[q]

    for b in range(nbuf):
      start(b, b)

    def body(t, _):
      for b in range(nbuf):
        c = nbuf * t + b
        wait(b)
        accum(c, b)

        @pl.when(c + nbuf < cpw)
        def _():
          start(c + nbuf, b)

      return 0

    lax.fori_loop(0, cpw // nbuf, body, 0)
    pltpu.sync_copy(pooled_v, out_hbm.at[pl.ds(base, spw)])

  return k(x_flat, table)


def _tc_head(pooled, wt, b2):
  bm = 1024

  def body(p_ref, w_ref, b_ref, o_ref):
    acc = jnp.dot(p_ref[...], w_ref[...], preferred_element_type=jnp.float32)
    o_ref[...] = acc * (1.0 / L) + b_ref[...]

  return pl.pallas_call(
      body,
      grid=(B // bm,),
      in_specs=[
          pl.BlockSpec((bm, D), lambda i: (i, 0)),
          pl.BlockSpec((D, NCLS), lambda i: (0, 0)),
          pl.BlockSpec((1, NCLS), lambda i: (0, 0)),
      ],
      out_specs=pl.BlockSpec((bm, NCLS), lambda i: (i, 0)),
      out_shape=jax.ShapeDtypeStruct((B, NCLS), jnp.float32),
  )(pooled, wt, b2)


def kernel(x, table, W, b):
  x32 = x.astype(jnp.int32)
  x_pad = jnp.pad(x32, ((0, 0), (0, LPAD - L)))
  pooled = _sc_pool(x_pad, table)
  return _tc_head(pooled, W.T, b.reshape(1, NCLS))


# speedup vs baseline: 2.1782x; 2.1782x over previous
import functools

import jax
import jax.numpy as jnp
from jax import lax
from jax.experimental import pallas as pl
from jax.experimental.pallas import tpu as pltpu
from jax.experimental.pallas import tpu_sc as plsc

B = 16384
L = 50
LPAD = 56
D = 64
NCLS = 100


def _sc_pool(x_pad, table):
  info = plsc.get_sparse_core_info()
  nc, ns = info.num_cores, info.num_subcores
  nw = nc * ns
  spw = B // nw
  cpw = spw
  nbuf = 4
  span = L
  x_flat = x_pad.reshape(nw, spw * LPAD)
  mesh = plsc.VectorSubcoreMesh(core_axis_name="c", subcore_axis_name="s")

  @functools.partial(
      pl.kernel,
      out_type=jax.ShapeDtypeStruct((B, D), jnp.float32),
      mesh=mesh,
      scratch_types=[
          pltpu.VMEM((spw * LPAD,), jnp.int32),
          pltpu.VMEM((nbuf, span, D), jnp.float32),
          pltpu.VMEM((spw, D), jnp.float32),
          pltpu.SemaphoreType.DMA((nbuf,)),
      ],
      compiler_params=pltpu.CompilerParams(use_tc_tiling_on_sc=False),
  )
  def k(x_hbm, table_hbm, out_hbm, idx_v, rows_v, pooled_v, sems):
    wid = lax.axis_index("s") * nc + lax.axis_index("c")
    base = wid * spw
    pltpu.sync_copy(x_hbm.at[wid], idx_v)

    def start(c, b):
      off = pl.multiple_of(c * LPAD, 8)
      pltpu.async_copy(
          table_hbm.at[idx_v.at[pl.ds(off, span)]], rows_v.at[b], sems.at[b])

    def wait(b):
      pltpu.make_async_copy(
          table_hbm.at[pl.ds(0, span)], rows_v.at[b], sems.at[b]).wait()

    def accum(c, b):
      accs = [rows_v[b, 0, pl.ds(16 * q, 16)] for q in range(D // 16)]
      for j in range(1, L):
        for q in range(D // 16):
          accs[q] = accs[q] + rows_v[b, j, pl.ds(16 * q, 16)]
      for q in range(D // 16):
        pooled_v[c, pl.ds(16 * q, 16)] = accs[q]

    for b in range(nbuf):
      start(b, b)

    def body(t, _):
      for b in range(nbuf):
        c = nbuf * t + b
        wait(b)
        accum(c, b)

        @pl.when(c + nbuf < cpw)
        def _():
          start(c + nbuf, b)

      return 0

    lax.fori_loop(0, cpw // nbuf, body, 0)
    pltpu.sync_copy(pooled_v, out_hbm.at[pl.ds(base, spw)])

  return k(x_flat, table)


def _tc_head(pooled, wt, b2):
  bm = 1024

  def body(p_ref, w_ref, b_ref, o_ref):
    acc = jnp.dot(p_ref[...], w_ref[...], preferred_element_type=jnp.float32)
    o_ref[...] = acc * (1.0 / L) + b_ref[...]

  return pl.pallas_call(
      body,
      grid=(B // bm,),
      in_specs=[
          pl.BlockSpec((bm, D), lambda i: (i, 0)),
          pl.BlockSpec((D, NCLS), lambda i: (0, 0)),
          pl.BlockSpec((1, NCLS), lambda i: (0, 0)),
      ],
      out_specs=pl.BlockSpec((bm, NCLS), lambda i: (i, 0)),
      out_shape=jax.ShapeDtypeStruct((B, NCLS), jnp.float32),
  )(pooled, wt, b2)


def kernel(x, table, W, b):
  x32 = x.astype(jnp.int32)
  x_pad = jnp.pad(x32, ((0, 0), (0, LPAD - L)))
  pooled = _sc_pool(x_pad, table)
  return _tc_head(pooled, W.T, b.reshape(1, NCLS))
